# baseline (speedup 1.0000x reference)
import functools

import jax
import jax.numpy as jnp
from jax import lax
from jax.experimental import pallas as pl
from jax.experimental.pallas import tpu as pltpu
from jax.experimental.pallas import tpu_sc as plsc

_NUM_Q = 100000
_D = 200
_B = 4096
_L = 50
_N = _B * _L
_DB = _D - 128

_info = plsc.get_sparse_core_info()
_NC = _info.num_cores
_NS = _info.num_subcores
_NW = _NC * _NS
_CH = 128
_NBUF = 3
_PER_W = _N // _NW
_STEPS = _PER_W // _CH
_GROUPS = _STEPS // _NBUF
_TAIL = _STEPS - _GROUPS * _NBUF

_mesh = plsc.VectorSubcoreMesh(core_axis_name="c", subcore_axis_name="s")


@functools.partial(
    pl.kernel,
    out_type=(
        jax.ShapeDtypeStruct((_N, 128), jnp.float32),
        jax.ShapeDtypeStruct((_N, _DB), jnp.float32),
    ),
    mesh=_mesh,
    scratch_types=[
        pltpu.VMEM((1, _STEPS, _CH), jnp.int32),
        pltpu.VMEM((_CH, 128), jnp.float32),
        pltpu.VMEM((_CH, 128), jnp.float32),
        pltpu.VMEM((_CH, 128), jnp.float32),
        pltpu.VMEM((_CH, 128), jnp.float32),
        pltpu.VMEM((_CH, 128), jnp.float32),
        pltpu.VMEM((_CH, 128), jnp.float32),
        pltpu.SemaphoreType.DMA,
        pltpu.SemaphoreType.DMA,
        pltpu.SemaphoreType.DMA,
        pltpu.SemaphoreType.DMA,
        pltpu.SemaphoreType.DMA,
        pltpu.SemaphoreType.DMA,
    ],
    compiler_params=pltpu.CompilerParams(use_tc_tiling_on_sc=False),
)
def _gather(tbla_hbm, tblb_hbm, idx_hbm, outa_hbm, outb_hbm, idx_v,
            a0, a1, a2, b0, b1, b2, g0, g1, g2, s0, s1, s2):
    bufa = (a0, a1, a2)
    bufb = (b0, b1, b2)
    gsem = (g0, g1, g2)
    ssem = (s0, s1, s2)
    wid = lax.axis_index("s") * _NC + lax.axis_index("c")
    base = wid * _PER_W
    pltpu.sync_copy(idx_hbm.at[pl.ds(wid, 1)], idx_v)

    def start_gathers(j, b):
        isl = idx_v.at[0, j]
        pltpu.async_copy(tbla_hbm.at[isl], bufa[b], gsem[b])
        pltpu.async_copy(tblb_hbm.at[isl], bufb[b], gsem[b])

    def wait_gathers(b):
        pltpu.make_async_copy(tbla_hbm.at[pl.ds(0, _CH)], bufa[b],
                              gsem[b]).wait()
        pltpu.make_async_copy(tblb_hbm.at[pl.ds(0, _CH)], bufb[b],
                              gsem[b]).wait()

    def start_stores(j, b):
        off = base + j * _CH
        pltpu.async_copy(bufa[b], outa_hbm.at[pl.ds(off, _CH)], ssem[b])
        pltpu.async_copy(bufb[b].at[:, pl.ds(0, _DB)],
                         outb_hbm.at[pl.ds(off, _CH)], ssem[b])

    def wait_stores(b):
        pltpu.make_async_copy(bufa[b], outa_hbm.at[pl.ds(0, _CH)],
                              ssem[b]).wait()
        pltpu.make_async_copy(bufb[b].at[:, pl.ds(0, _DB)],
                              outb_hbm.at[pl.ds(0, _CH)], ssem[b]).wait()

    for b in range(_NBUF):
        start_gathers(b, b)

    def group(g, carry):
        j0 = g * _NBUF
        for b in range(_NBUF):
            wait_gathers(b)
            start_stores(j0 + b, b)

            @pl.when(j0 + b + _NBUF < _STEPS)
            def _():
                wait_stores(b)
                start_gathers(j0 + b + _NBUF, b)
        return carry

    lax.fori_loop(0, _GROUPS, group, 0)
    for t in range(_TAIL):
        j = _GROUPS * _NBUF + t
        b = j % _NBUF
        wait_gathers(b)
        start_stores(j, b)
    for b in range(_NBUF):
        wait_stores(b)


def kernel(indices, table):
    tbla = table[:, :128]
    tblb = jnp.pad(table[:, 128:], ((0, 0), (0, 128 - _DB)))
    idx2 = indices.reshape(_NW, _STEPS, _CH)
    outa, outb = _gather(tbla, tblb, idx2)
    return jnp.concatenate([outa, outb], axis=1).reshape(_B, _L, _D)


# speedup vs baseline: 1.1129x; 1.1129x over previous
import functools

import jax
import jax.numpy as jnp
from jax import lax
from jax.experimental import pallas as pl
from jax.experimental.pallas import tpu as pltpu
from jax.experimental.pallas import tpu_sc as plsc

_NUM_Q = 100000
_D = 200
_B = 4096
_L = 50
_N = _B * _L
_DB = _D - 128

_info = plsc.get_sparse_core_info()
_NC = _info.num_cores
_NS = _info.num_subcores
_NW = _NC * _NS
_CH = 128
_NBUF = 3
_PER_W = _N // _NW
_STEPS = _PER_W // _CH
_GROUPS = _STEPS // _NBUF
_TAIL = _STEPS - _GROUPS * _NBUF

_mesh = plsc.VectorSubcoreMesh(core_axis_name="c", subcore_axis_name="s")


@functools.partial(
    pl.kernel,
    out_type=(
        jax.ShapeDtypeStruct((_N, 128), jnp.float32),
        jax.ShapeDtypeStruct((_N, 128), jnp.float32),
    ),
    mesh=_mesh,
    scratch_types=[
        pltpu.VMEM((1, _STEPS, _CH), jnp.int32),
        pltpu.VMEM((_CH, 128), jnp.float32),
        pltpu.VMEM((_CH, 128), jnp.float32),
        pltpu.VMEM((_CH, 128), jnp.float32),
        pltpu.VMEM((_CH, 128), jnp.float32),
        pltpu.VMEM((_CH, 128), jnp.float32),
        pltpu.VMEM((_CH, 128), jnp.float32),
        pltpu.SemaphoreType.DMA,
        pltpu.SemaphoreType.DMA,
        pltpu.SemaphoreType.DMA,
        pltpu.SemaphoreType.DMA,
        pltpu.SemaphoreType.DMA,
        pltpu.SemaphoreType.DMA,
    ],
    compiler_params=pltpu.CompilerParams(use_tc_tiling_on_sc=False),
)
def _gather(tbla_hbm, tblb_hbm, idx_hbm, outa_hbm, outb_hbm, idx_v,
            a0, a1, a2, b0, b1, b2, g0, g1, g2, s0, s1, s2):
    bufa = (a0, a1, a2)
    bufb = (b0, b1, b2)
    gsem = (g0, g1, g2)
    ssem = (s0, s1, s2)
    wid = lax.axis_index("s") * _NC + lax.axis_index("c")
    base = wid * _PER_W
    pltpu.sync_copy(idx_hbm.at[pl.ds(wid, 1)], idx_v)

    def start_gathers(j, b):
        isl = idx_v.at[0, j]
        pltpu.async_copy(tbla_hbm.at[isl], bufa[b], gsem[b])
        pltpu.async_copy(tblb_hbm.at[isl], bufb[b], gsem[b])

    def wait_gathers(b):
        pltpu.make_async_copy(tbla_hbm.at[pl.ds(0, _CH)], bufa[b],
                              gsem[b]).wait()
        pltpu.make_async_copy(tblb_hbm.at[pl.ds(0, _CH)], bufb[b],
                              gsem[b]).wait()

    def start_stores(j, b):
        off = base + j * _CH
        pltpu.async_copy(bufa[b], outa_hbm.at[pl.ds(off, _CH)], ssem[b])
        pltpu.async_copy(bufb[b], outb_hbm.at[pl.ds(off, _CH)], ssem[b])

    def wait_stores(b):
        pltpu.make_async_copy(bufa[b], outa_hbm.at[pl.ds(0, _CH)],
                              ssem[b]).wait()
        pltpu.make_async_copy(bufb[b], outb_hbm.at[pl.ds(0, _CH)],
                              ssem[b]).wait()

    for b in range(_NBUF):
        start_gathers(b, b)

    def group(g, carry):
        j0 = g * _NBUF
        for b in range(_NBUF):
            wait_gathers(b)
            start_stores(j0 + b, b)

            @pl.when(j0 + b + _NBUF < _STEPS)
            def _():
                wait_stores(b)
                start_gathers(j0 + b + _NBUF, b)
        return carry

    lax.fori_loop(0, _GROUPS, group, 0)
    for t in range(_TAIL):
        j = _GROUPS * _NBUF + t
        b = j % _NBUF
        wait_gathers(b)
        start_stores(j, b)
    for b in range(_NBUF):
        wait_stores(b)


def kernel(indices, table):
    tbla = table[:, :128]
    tblb = jnp.pad(table[:, 128:], ((0, 0), (0, 128 - _DB)))
    idx2 = indices.reshape(_NW, _STEPS, _CH)
    outa, outb = _gather(tbla, tblb, idx2)
    return jnp.concatenate([outa, outb[:, :_DB]], axis=1).reshape(
        _B, _L, _D)
